# Initial kernel scaffold; baseline (speedup 1.0000x reference)
import jax
import jax.numpy as jnp
from jax.experimental import pallas as pl


def kernel(boxes, scores):
    raise NotImplementedError("write your pallas kernel here")


# speedup vs baseline: 39.4478x; 39.4478x over previous
import functools

import jax
import jax.numpy as jnp
from jax.experimental import pallas as pl
from jax.experimental.pallas import tpu as pltpu

_N = 5000
_R = 8
_C = 640
_NP = _R * _C
_THRESH = 0.3


def _index_grid():
    rr = jax.lax.broadcasted_iota(jnp.int32, (_R, _C), 0)
    cc = jax.lax.broadcasted_iota(jnp.int32, (_R, _C), 1)
    return rr * _C + cc


def _corners(in_ref):
    cx = in_ref[0]
    cy = in_ref[1]
    w = in_ref[2]
    h = in_ref[3]
    return cx - w / 2.0, cy - h / 2.0, cx + w / 2.0, cy + h / 2.0


def _select(s, maskf, idx):
    m = jnp.max(jnp.where(maskf > 0.5, s, -1.0))
    i = jnp.min(jnp.where((maskf > 0.5) & (s == m), idx, _NP))
    return m, i


def _nms_kernel(in_ref, out_ref, rem_ref):
    idx = _index_grid()
    valid = (idx < _N).astype(jnp.float32)
    rem_ref[...] = valid
    out_ref[...] = jnp.zeros((5, _R, _C), jnp.float32)

    m0, i0 = _select(in_ref[4], valid, idx)

    def cond(carry):
        return carry[0] > -0.5

    def body(carry):
        m, i = carry
        idx = _index_grid()
        x1, y1, x2, y2 = _corners(in_ref)
        s = in_ref[4]
        rem = rem_ref[...]

        oh = idx == i
        ohf = oh.astype(jnp.float32)
        xi1 = jnp.sum(x1 * ohf)
        yi1 = jnp.sum(y1 * ohf)
        xi2 = jnp.sum(x2 * ohf)
        yi2 = jnp.sum(y2 * ohf)
        ai = jnp.maximum(xi2 - xi1, 0.0) * jnp.maximum(yi2 - yi1, 0.0)

        area = jnp.maximum(x2 - x1, 0.0) * jnp.maximum(y2 - y1, 0.0)
        xx1 = jnp.maximum(x1, xi1)
        yy1 = jnp.maximum(y1, yi1)
        xx2 = jnp.minimum(x2, xi2)
        yy2 = jnp.minimum(y2, yi2)
        inter = jnp.maximum(xx2 - xx1, 0.0) * jnp.maximum(yy2 - yy1, 0.0)
        union = area + ai - inter
        iou = inter / (union + 1e-9)

        newrem = jnp.where((iou > _THRESH) | oh, 0.0, rem)
        rem_ref[...] = newrem

        rank = jnp.sum((s > m).astype(jnp.int32)) + jnp.sum(
            ((s == m) & (idx < i)).astype(jnp.int32)
        )
        ohr = (idx == rank).astype(jnp.float32)
        out_ref[0] = out_ref[0] + m * ohr
        out_ref[1] = out_ref[1] + xi1 * ohr
        out_ref[2] = out_ref[2] + yi1 * ohr
        out_ref[3] = out_ref[3] + xi2 * ohr
        out_ref[4] = out_ref[4] + yi2 * ohr

        return _select(s, newrem, idx)

    jax.lax.while_loop(cond, body, (m0, i0))


@functools.partial(jax.jit, static_argnames=())
def kernel(boxes, scores):
    bp = jnp.zeros((_NP, 4), jnp.float32).at[:_N].set(boxes)
    sp = jnp.full((_NP,), -1.0, jnp.float32).at[:_N].set(scores)
    pack = jnp.concatenate([bp.T, sp[None, :]], axis=0).reshape(5, _R, _C)
    out = pl.pallas_call(
        _nms_kernel,
        out_shape=jax.ShapeDtypeStruct((5, _R, _C), jnp.float32),
        scratch_shapes=[pltpu.VMEM((_R, _C), jnp.float32)],
    )(pack)
    return out.reshape(5, _NP)[:, :_N].T
